# baseline (speedup 1.0000x reference)
import functools

import jax
import jax.numpy as jnp
from jax import lax
from jax.experimental import pallas as pl
from jax.experimental.pallas import tpu as pltpu

C = 64
C2 = 2 * C
EPS = 1e-5
NEG_SLOPE = 0.2
BN_CONV = 4
BN_RES = 8
TH_RES = 16


def _zero_borders(xp_ref):
    n, hp2, wp2, _ = xp_ref.shape
    w2 = wp2 - 1
    cdt = xp_ref.dtype
    xp_ref[:, :, 0:1, 0:C] = jnp.zeros((n, hp2, 1, C), cdt)
    xp_ref[:, :, w2:wp2, C:C2] = jnp.zeros((n, hp2, 1, C), cdt)
    xp_ref[:, 0:1, :, :] = jnp.zeros((n, 1, wp2, C2), cdt)
    xp_ref[:, hp2 - 1:hp2, :, :] = jnp.zeros((n, 1, wp2, C2), cdt)


def _pack_weights(w_ref, wsc_ref):
    wsc_ref[...] = jnp.zeros(wsc_ref.shape, wsc_ref.dtype)
    for ky in range(3):
        for kx in range(3):
            blk = w_ref[3 * ky + kx].astype(wsc_ref.dtype)
            wsc_ref[(4 * ky + kx) * C:(4 * ky + kx + 1) * C, 0:C] = blk
            wsc_ref[(4 * ky + kx + 1) * C:(4 * ky + kx + 2) * C, C:C2] = blk


def _bn_scale_shift(stat_ref, g_ref, b_ref, m_total):
    s = jnp.sum(stat_ref[...], axis=0)
    total = s[0:1, 0:C] + s[0:1, C:C2]
    totsq = s[1:2, 0:C] + s[1:2, C:C2]
    mean = total * (1.0 / m_total)
    var = jnp.maximum(totsq * (1.0 / m_total) - mean * mean, 0.0)
    scale = g_ref[...] * lax.rsqrt(var + EPS)
    shift = b_ref[...] - mean * scale
    return scale, shift


def _six_dots(xp_ref, wsc_ref, out_ref, stat_ref):
    n, hp2, wp2, _ = xp_ref.shape
    h, w2 = hp2 - 2, wp2 - 1
    m2 = n * h * w2
    acc = None
    for ky in range(3):
        for s in range(2):
            stripe = xp_ref[:, ky:ky + h, s:s + w2, :].reshape(m2, C2)
            blk = wsc_ref[(2 * ky + s) * C2:(2 * ky + s + 1) * C2, :]
            d = jnp.dot(stripe, blk, preferred_element_type=jnp.float32)
            acc = d if acc is None else acc + d
    ob = acc.reshape(n, h, w2, C2).astype(out_ref.dtype)
    out_ref[...] = ob
    o2 = ob.reshape(m2, C2)
    ones = jnp.ones((1, m2), o2.dtype)
    s1 = jnp.dot(ones, o2, preferred_element_type=jnp.float32)
    gram = lax.dot_general(o2, o2, (((0,), (0,)), ((), ())),
                           preferred_element_type=jnp.float32)
    row = lax.broadcasted_iota(jnp.int32, (C2, C2), 0)
    col = lax.broadcasted_iota(jnp.int32, (C2, C2), 1)
    s2 = jnp.sum(jnp.where(row == col, gram, 0.0), axis=0, keepdims=True)
    stat_ref[...] = jnp.concatenate([s1, s2], axis=0).reshape(1, 2, C2)


def _conv1_kernel(x_ref, w_ref, out_ref, stat_ref, xp_ref, wsc_ref):
    _pack_weights(w_ref, wsc_ref)
    v = x_ref[...].astype(xp_ref.dtype)
    n, _, h, wd = v.shape
    w2 = wd // 2
    t = jnp.transpose(v, (0, 2, 3, 1))
    t = t.reshape(n, h, w2, 2, C)
    wp2 = xp_ref.shape[2]
    xp_ref[:, 1:1 + h, 0:w2, C:C2] = t[:, :, :, 0, :]
    xp_ref[:, 1:1 + h, 1:wp2, 0:C] = t[:, :, :, 1, :]
    _zero_borders(xp_ref)
    _six_dots(xp_ref, wsc_ref, out_ref, stat_ref)


def _conv2_kernel(stat1_ref, g_ref, b_ref, y_ref, w_ref, out_ref, stat_ref,
                  xp_ref, wsc_ref, *, m_total):
    _pack_weights(w_ref, wsc_ref)
    scale, shift = _bn_scale_shift(stat1_ref, g_ref, b_ref, m_total)
    v = y_ref[...]
    wp2 = xp_ref.shape[2]
    h, w2 = v.shape[1], v.shape[2]

    def act(half):
        a = half.astype(jnp.float32) * scale + shift
        return jnp.where(a > 0, a, NEG_SLOPE * a).astype(xp_ref.dtype)

    xp_ref[:, 1:1 + h, 0:w2, C:C2] = act(v[..., 0:C])
    xp_ref[:, 1:1 + h, 1:wp2, 0:C] = act(v[..., C:C2])
    _zero_borders(xp_ref)
    _six_dots(xp_ref, wsc_ref, out_ref, stat_ref)


def _resid_kernel(stat2_ref, g_ref, b_ref, x_ref, y_ref, o_ref, *, m_total):
    scale, shift = _bn_scale_shift(stat2_ref, g_ref, b_ref, m_total)
    y = y_ref[...]
    n, th, w2, _ = y.shape
    ze = (y[..., 0:C].astype(jnp.float32) * scale + shift)
    zo = (y[..., C:C2].astype(jnp.float32) * scale + shift)
    zw = jnp.concatenate([ze.reshape(n, th, w2, 1, C),
                          zo.reshape(n, th, w2, 1, C)],
                         axis=3).reshape(n, th, 2 * w2, C)
    o_ref[...] = x_ref[...] + jnp.transpose(zw, (0, 3, 1, 2))


def _conv1_call(x, w1, *, N, H, W2, out_dtype):
    W = 2 * W2
    nb = N // BN_CONV
    return pl.pallas_call(
        _conv1_kernel,
        grid=(nb,),
        in_specs=[pl.BlockSpec((BN_CONV, C, H, W), lambda i: (i, 0, 0, 0)),
                  pl.BlockSpec((9, C, C), lambda i: (0, 0, 0))],
        out_specs=(pl.BlockSpec((BN_CONV, H, W2, C2), lambda i: (i, 0, 0, 0)),
                   pl.BlockSpec((1, 2, C2), lambda i: (i, 0, 0))),
        out_shape=(jax.ShapeDtypeStruct((N, H, W2, C2), out_dtype),
                   jax.ShapeDtypeStruct((nb, 2, C2), jnp.float32)),
        scratch_shapes=[pltpu.VMEM((BN_CONV, H + 2, W2 + 1, C2), jnp.bfloat16),
                        pltpu.VMEM((12 * C, C2), jnp.bfloat16)],
        compiler_params=pltpu.CompilerParams(
            dimension_semantics=("parallel",),
            vmem_limit_bytes=100 * 1024 * 1024),
    )(x, w1)


def _conv2_call(src, st1, w2r, g1, be1, *, N, H, W2, m_total, out_dtype):
    nb = N // BN_CONV
    vec = pl.BlockSpec((1, C), lambda i: (0, 0))
    stat_in = pl.BlockSpec(st1.shape, lambda i: (0, 0, 0))
    return pl.pallas_call(
        functools.partial(_conv2_kernel, m_total=m_total),
        grid=(nb,),
        in_specs=[stat_in, vec, vec,
                  pl.BlockSpec((BN_CONV, H, W2, C2), lambda i: (i, 0, 0, 0)),
                  pl.BlockSpec((9, C, C), lambda i: (0, 0, 0))],
        out_specs=(pl.BlockSpec((BN_CONV, H, W2, C2), lambda i: (i, 0, 0, 0)),
                   pl.BlockSpec((1, 2, C2), lambda i: (i, 0, 0))),
        out_shape=(jax.ShapeDtypeStruct((N, H, W2, C2), out_dtype),
                   jax.ShapeDtypeStruct((nb, 2, C2), jnp.float32)),
        scratch_shapes=[pltpu.VMEM((BN_CONV, H + 2, W2 + 1, C2), jnp.bfloat16),
                        pltpu.VMEM((12 * C, C2), jnp.bfloat16)],
        compiler_params=pltpu.CompilerParams(
            dimension_semantics=("parallel",),
            vmem_limit_bytes=100 * 1024 * 1024),
    )(st1, g1, be1, src, w2r)


def _resid_call(x, y, st2, g2, be2, *, N, H, W2, m_total):
    W = 2 * W2
    nbn, nbh = N // BN_RES, H // TH_RES
    nchw = pl.BlockSpec((BN_RES, C, TH_RES, W), lambda ni, hi: (ni, 0, hi, 0))
    band = pl.BlockSpec((BN_RES, TH_RES, W2, C2), lambda ni, hi: (ni, hi, 0, 0))
    vec = pl.BlockSpec((1, C), lambda ni, hi: (0, 0))
    stat_in = pl.BlockSpec(st2.shape, lambda ni, hi: (0, 0, 0))
    return pl.pallas_call(
        functools.partial(_resid_kernel, m_total=m_total),
        grid=(nbn, nbh),
        in_specs=[stat_in, vec, vec, nchw, band],
        out_specs=nchw,
        out_shape=jax.ShapeDtypeStruct((N, C, H, W), jnp.float32),
        compiler_params=pltpu.CompilerParams(
            dimension_semantics=("parallel", "parallel")),
    )(st2, g2, be2, x, y)


def kernel(x, w1, b1, g1, be1, w2, b2, g2, be2):
    del b1, b2
    x = jnp.asarray(x, jnp.float32)
    N, _, H, W = x.shape
    W2 = W // 2
    M = float(N * H * W)
    g1 = jnp.asarray(g1, jnp.float32).reshape(1, C)
    be1 = jnp.asarray(be1, jnp.float32).reshape(1, C)
    g2 = jnp.asarray(g2, jnp.float32).reshape(1, C)
    be2 = jnp.asarray(be2, jnp.float32).reshape(1, C)
    w1r = jnp.asarray(w1, jnp.float32).reshape(9, C, C)
    w2r = jnp.asarray(w2, jnp.float32).reshape(9, C, C)

    y1, st1 = _conv1_call(x, w1r, N=N, H=H, W2=W2, out_dtype=jnp.bfloat16)
    y2, st2 = _conv2_call(y1, st1, w2r, g1, be1, N=N, H=H, W2=W2,
                          m_total=M, out_dtype=jnp.bfloat16)
    return _resid_call(x, y2, st2, g2, be2, N=N, H=H, W2=W2, m_total=M)


# speedup vs baseline: 1.1087x; 1.1087x over previous
import functools

import jax
import jax.numpy as jnp
from jax import lax
from jax.experimental import pallas as pl
from jax.experimental.pallas import tpu as pltpu

C = 64
C2 = 2 * C
EPS = 1e-5
NEG_SLOPE = 0.2
BN_CONV = 4
BN_RES = 8
TH_RES = 16


def _zero_borders(xp_ref):
    n, hp2, wp2, _ = xp_ref.shape
    w2 = wp2 - 1
    cdt = xp_ref.dtype
    xp_ref[:, :, 0:1, 0:C] = jnp.zeros((n, hp2, 1, C), cdt)
    xp_ref[:, :, w2:wp2, C:C2] = jnp.zeros((n, hp2, 1, C), cdt)
    xp_ref[:, 0:1, :, :] = jnp.zeros((n, 1, wp2, C2), cdt)
    xp_ref[:, hp2 - 1:hp2, :, :] = jnp.zeros((n, 1, wp2, C2), cdt)


def _pack_weights(w_ref, wsc_ref):
    wsc_ref[...] = jnp.zeros(wsc_ref.shape, wsc_ref.dtype)
    for ky in range(3):
        for kx in range(3):
            blk = w_ref[3 * ky + kx].astype(wsc_ref.dtype)
            wsc_ref[(4 * ky + kx) * C:(4 * ky + kx + 1) * C, 0:C] = blk
            wsc_ref[(4 * ky + kx + 1) * C:(4 * ky + kx + 2) * C, C:C2] = blk


def _bn_scale_shift(stat_ref, g_ref, b_ref, m_total):
    s = jnp.sum(stat_ref[...], axis=0)
    total = s[0:1, 0:C] + s[0:1, C:C2]
    totsq = s[1:2, 0:C] + s[1:2, C:C2]
    mean = total * (1.0 / m_total)
    var = jnp.maximum(totsq * (1.0 / m_total) - mean * mean, 0.0)
    scale = g_ref[...] * lax.rsqrt(var + EPS)
    shift = b_ref[...] - mean * scale
    return scale, shift


def _six_dots(xp_ref, wsc_ref, out_ref, stat_ref):
    n, hp2, wp2, _ = xp_ref.shape
    h, w2 = hp2 - 2, wp2 - 1
    m2 = n * h * w2
    col = jnp.concatenate(
        [xp_ref[:, ky:ky + h, s:s + w2, :].reshape(m2, C2)
         for ky in range(3) for s in range(2)], axis=1)
    acc = jnp.dot(col, wsc_ref[...], preferred_element_type=jnp.float32)
    out_ref[...] = acc.reshape(n, h, w2, C2).astype(out_ref.dtype)
    s1 = jnp.sum(acc, axis=0, keepdims=True)
    s2 = jnp.sum(acc * acc, axis=0, keepdims=True)
    stat_ref[...] = jnp.concatenate([s1, s2], axis=0).reshape(1, 2, C2)


def _conv1_kernel(x_ref, w_ref, out_ref, stat_ref, xp_ref, wsc_ref):
    _pack_weights(w_ref, wsc_ref)
    v = x_ref[...].astype(xp_ref.dtype)
    n, _, h, wd = v.shape
    w2 = wd // 2
    t = jnp.transpose(v, (0, 2, 3, 1))
    t = t.reshape(n, h, w2, 2, C)
    wp2 = xp_ref.shape[2]
    xp_ref[:, 1:1 + h, 0:w2, C:C2] = t[:, :, :, 0, :]
    xp_ref[:, 1:1 + h, 1:wp2, 0:C] = t[:, :, :, 1, :]
    _zero_borders(xp_ref)
    _six_dots(xp_ref, wsc_ref, out_ref, stat_ref)


def _conv2_kernel(stat1_ref, g_ref, b_ref, y_ref, w_ref, out_ref, stat_ref,
                  xp_ref, wsc_ref, *, m_total):
    _pack_weights(w_ref, wsc_ref)
    scale, shift = _bn_scale_shift(stat1_ref, g_ref, b_ref, m_total)
    v = y_ref[...]
    wp2 = xp_ref.shape[2]
    h, w2 = v.shape[1], v.shape[2]

    def act(half):
        a = half.astype(jnp.float32) * scale + shift
        return jnp.where(a > 0, a, NEG_SLOPE * a).astype(xp_ref.dtype)

    xp_ref[:, 1:1 + h, 0:w2, C:C2] = act(v[..., 0:C])
    xp_ref[:, 1:1 + h, 1:wp2, 0:C] = act(v[..., C:C2])
    _zero_borders(xp_ref)
    _six_dots(xp_ref, wsc_ref, out_ref, stat_ref)


def _resid_kernel(stat2_ref, g_ref, b_ref, x_ref, y_ref, o_ref, *, m_total):
    scale, shift = _bn_scale_shift(stat2_ref, g_ref, b_ref, m_total)
    y = y_ref[...]
    n, th, w2, _ = y.shape
    ze = (y[..., 0:C].astype(jnp.float32) * scale + shift)
    zo = (y[..., C:C2].astype(jnp.float32) * scale + shift)
    zw = jnp.concatenate([ze.reshape(n, th, w2, 1, C),
                          zo.reshape(n, th, w2, 1, C)],
                         axis=3).reshape(n, th, 2 * w2, C)
    o_ref[...] = x_ref[...] + jnp.transpose(zw, (0, 3, 1, 2))


def _conv1_call(x, w1, *, N, H, W2, out_dtype):
    W = 2 * W2
    nb = N // BN_CONV
    return pl.pallas_call(
        _conv1_kernel,
        grid=(nb,),
        in_specs=[pl.BlockSpec((BN_CONV, C, H, W), lambda i: (i, 0, 0, 0)),
                  pl.BlockSpec((9, C, C), lambda i: (0, 0, 0))],
        out_specs=(pl.BlockSpec((BN_CONV, H, W2, C2), lambda i: (i, 0, 0, 0)),
                   pl.BlockSpec((1, 2, C2), lambda i: (i, 0, 0))),
        out_shape=(jax.ShapeDtypeStruct((N, H, W2, C2), out_dtype),
                   jax.ShapeDtypeStruct((nb, 2, C2), jnp.float32)),
        scratch_shapes=[pltpu.VMEM((BN_CONV, H + 2, W2 + 1, C2), jnp.bfloat16),
                        pltpu.VMEM((12 * C, C2), jnp.bfloat16)],
        compiler_params=pltpu.CompilerParams(
            dimension_semantics=("parallel",),
            vmem_limit_bytes=100 * 1024 * 1024),
    )(x, w1)


def _conv2_call(src, st1, w2r, g1, be1, *, N, H, W2, m_total, out_dtype):
    nb = N // BN_CONV
    vec = pl.BlockSpec((1, C), lambda i: (0, 0))
    stat_in = pl.BlockSpec(st1.shape, lambda i: (0, 0, 0))
    return pl.pallas_call(
        functools.partial(_conv2_kernel, m_total=m_total),
        grid=(nb,),
        in_specs=[stat_in, vec, vec,
                  pl.BlockSpec((BN_CONV, H, W2, C2), lambda i: (i, 0, 0, 0)),
                  pl.BlockSpec((9, C, C), lambda i: (0, 0, 0))],
        out_specs=(pl.BlockSpec((BN_CONV, H, W2, C2), lambda i: (i, 0, 0, 0)),
                   pl.BlockSpec((1, 2, C2), lambda i: (i, 0, 0))),
        out_shape=(jax.ShapeDtypeStruct((N, H, W2, C2), out_dtype),
                   jax.ShapeDtypeStruct((nb, 2, C2), jnp.float32)),
        scratch_shapes=[pltpu.VMEM((BN_CONV, H + 2, W2 + 1, C2), jnp.bfloat16),
                        pltpu.VMEM((12 * C, C2), jnp.bfloat16)],
        compiler_params=pltpu.CompilerParams(
            dimension_semantics=("parallel",),
            vmem_limit_bytes=100 * 1024 * 1024),
    )(st1, g1, be1, src, w2r)


def _resid_call(x, y, st2, g2, be2, *, N, H, W2, m_total):
    W = 2 * W2
    nbn, nbh = N // BN_RES, H // TH_RES
    nchw = pl.BlockSpec((BN_RES, C, TH_RES, W), lambda ni, hi: (ni, 0, hi, 0))
    band = pl.BlockSpec((BN_RES, TH_RES, W2, C2), lambda ni, hi: (ni, hi, 0, 0))
    vec = pl.BlockSpec((1, C), lambda ni, hi: (0, 0))
    stat_in = pl.BlockSpec(st2.shape, lambda ni, hi: (0, 0, 0))
    return pl.pallas_call(
        functools.partial(_resid_kernel, m_total=m_total),
        grid=(nbn, nbh),
        in_specs=[stat_in, vec, vec, nchw, band],
        out_specs=nchw,
        out_shape=jax.ShapeDtypeStruct((N, C, H, W), jnp.float32),
        compiler_params=pltpu.CompilerParams(
            dimension_semantics=("parallel", "parallel")),
    )(st2, g2, be2, x, y)


def kernel(x, w1, b1, g1, be1, w2, b2, g2, be2):
    del b1, b2
    x = jnp.asarray(x, jnp.float32)
    N, _, H, W = x.shape
    W2 = W // 2
    M = float(N * H * W)
    g1 = jnp.asarray(g1, jnp.float32).reshape(1, C)
    be1 = jnp.asarray(be1, jnp.float32).reshape(1, C)
    g2 = jnp.asarray(g2, jnp.float32).reshape(1, C)
    be2 = jnp.asarray(be2, jnp.float32).reshape(1, C)
    w1r = jnp.asarray(w1, jnp.float32).reshape(9, C, C)
    w2r = jnp.asarray(w2, jnp.float32).reshape(9, C, C)

    y1, st1 = _conv1_call(x, w1r, N=N, H=H, W2=W2, out_dtype=jnp.bfloat16)
    y2, st2 = _conv2_call(y1, st1, w2r, g1, be1, N=N, H=H, W2=W2,
                          m_total=M, out_dtype=jnp.bfloat16)
    return _resid_call(x, y2, st2, g2, be2, N=N, H=H, W2=W2, m_total=M)


# speedup vs baseline: 1.1096x; 1.0009x over previous
import functools

import jax
import jax.numpy as jnp
from jax import lax
from jax.experimental import pallas as pl
from jax.experimental.pallas import tpu as pltpu

C = 64
C2 = 2 * C
EPS = 1e-5
NEG_SLOPE = 0.2
BN_CONV = 2
BN_RES = 8
TH_RES = 16


def _zero_borders(xp_ref):
    n, hp2, wp2, _ = xp_ref.shape
    w2 = wp2 - 1
    cdt = xp_ref.dtype
    xp_ref[:, :, 0:1, 0:C] = jnp.zeros((n, hp2, 1, C), cdt)
    xp_ref[:, :, w2:wp2, C:C2] = jnp.zeros((n, hp2, 1, C), cdt)
    xp_ref[:, 0:1, :, :] = jnp.zeros((n, 1, wp2, C2), cdt)
    xp_ref[:, hp2 - 1:hp2, :, :] = jnp.zeros((n, 1, wp2, C2), cdt)


def _pack_weights(w_ref, wsc_ref):
    wsc_ref[...] = jnp.zeros(wsc_ref.shape, wsc_ref.dtype)
    for ky in range(3):
        for kx in range(3):
            blk = w_ref[3 * ky + kx].astype(wsc_ref.dtype)
            wsc_ref[(4 * ky + kx) * C:(4 * ky + kx + 1) * C, 0:C] = blk
            wsc_ref[(4 * ky + kx + 1) * C:(4 * ky + kx + 2) * C, C:C2] = blk


def _bn_scale_shift(stat_ref, g_ref, b_ref, m_total):
    s = jnp.sum(stat_ref[...], axis=0)
    total = s[0:1, 0:C] + s[0:1, C:C2]
    totsq = s[1:2, 0:C] + s[1:2, C:C2]
    mean = total * (1.0 / m_total)
    var = jnp.maximum(totsq * (1.0 / m_total) - mean * mean, 0.0)
    scale = g_ref[...] * lax.rsqrt(var + EPS)
    shift = b_ref[...] - mean * scale
    return scale, shift


def _six_dots(xp_ref, wsc_ref, out_ref, stat_ref):
    n, hp2, wp2, _ = xp_ref.shape
    h, w2 = hp2 - 2, wp2 - 1
    m2 = n * h * w2
    col = jnp.concatenate(
        [xp_ref[:, ky:ky + h, s:s + w2, :].reshape(m2, C2)
         for ky in range(3) for s in range(2)], axis=1)
    acc = jnp.dot(col, wsc_ref[...], preferred_element_type=jnp.float32)
    out_ref[...] = acc.reshape(n, h, w2, C2).astype(out_ref.dtype)
    s1 = jnp.sum(acc, axis=0, keepdims=True)
    s2 = jnp.sum(acc * acc, axis=0, keepdims=True)
    stat_ref[...] = jnp.concatenate([s1, s2], axis=0).reshape(1, 2, C2)


def _conv1_kernel(x_ref, w_ref, out_ref, stat_ref, xp_ref, wsc_ref):
    _pack_weights(w_ref, wsc_ref)
    v = x_ref[...].astype(xp_ref.dtype)
    n, _, h, wd = v.shape
    w2 = wd // 2
    t = jnp.transpose(v, (0, 2, 3, 1))
    t = t.reshape(n, h, w2, 2, C)
    wp2 = xp_ref.shape[2]
    xp_ref[:, 1:1 + h, 0:w2, C:C2] = t[:, :, :, 0, :]
    xp_ref[:, 1:1 + h, 1:wp2, 0:C] = t[:, :, :, 1, :]
    _zero_borders(xp_ref)
    _six_dots(xp_ref, wsc_ref, out_ref, stat_ref)


def _conv2_kernel(stat1_ref, g_ref, b_ref, y_ref, w_ref, out_ref, stat_ref,
                  xp_ref, wsc_ref, *, m_total):
    _pack_weights(w_ref, wsc_ref)
    scale, shift = _bn_scale_shift(stat1_ref, g_ref, b_ref, m_total)
    v = y_ref[...]
    wp2 = xp_ref.shape[2]
    h, w2 = v.shape[1], v.shape[2]

    def act(half):
        a = half.astype(jnp.float32) * scale + shift
        return jnp.where(a > 0, a, NEG_SLOPE * a).astype(xp_ref.dtype)

    xp_ref[:, 1:1 + h, 0:w2, C:C2] = act(v[..., 0:C])
    xp_ref[:, 1:1 + h, 1:wp2, 0:C] = act(v[..., C:C2])
    _zero_borders(xp_ref)
    _six_dots(xp_ref, wsc_ref, out_ref, stat_ref)


def _resid_kernel(stat2_ref, g_ref, b_ref, x_ref, y_ref, o_ref, *, m_total):
    scale, shift = _bn_scale_shift(stat2_ref, g_ref, b_ref, m_total)
    y = y_ref[...]
    n, th, w2, _ = y.shape
    ze = (y[..., 0:C].astype(jnp.float32) * scale + shift)
    zo = (y[..., C:C2].astype(jnp.float32) * scale + shift)
    zw = jnp.concatenate([ze.reshape(n, th, w2, 1, C),
                          zo.reshape(n, th, w2, 1, C)],
                         axis=3).reshape(n, th, 2 * w2, C)
    o_ref[...] = x_ref[...] + jnp.transpose(zw, (0, 3, 1, 2))


def _conv1_call(x, w1, *, N, H, W2, out_dtype):
    W = 2 * W2
    nb = N // BN_CONV
    return pl.pallas_call(
        _conv1_kernel,
        grid=(nb,),
        in_specs=[pl.BlockSpec((BN_CONV, C, H, W), lambda i: (i, 0, 0, 0)),
                  pl.BlockSpec((9, C, C), lambda i: (0, 0, 0))],
        out_specs=(pl.BlockSpec((BN_CONV, H, W2, C2), lambda i: (i, 0, 0, 0)),
                   pl.BlockSpec((1, 2, C2), lambda i: (i, 0, 0))),
        out_shape=(jax.ShapeDtypeStruct((N, H, W2, C2), out_dtype),
                   jax.ShapeDtypeStruct((nb, 2, C2), jnp.float32)),
        scratch_shapes=[pltpu.VMEM((BN_CONV, H + 2, W2 + 1, C2), jnp.bfloat16),
                        pltpu.VMEM((12 * C, C2), jnp.bfloat16)],
        compiler_params=pltpu.CompilerParams(
            dimension_semantics=("parallel",),
            vmem_limit_bytes=100 * 1024 * 1024),
    )(x, w1)


def _conv2_call(src, st1, w2r, g1, be1, *, N, H, W2, m_total, out_dtype):
    nb = N // BN_CONV
    vec = pl.BlockSpec((1, C), lambda i: (0, 0))
    stat_in = pl.BlockSpec(st1.shape, lambda i: (0, 0, 0))
    return pl.pallas_call(
        functools.partial(_conv2_kernel, m_total=m_total),
        grid=(nb,),
        in_specs=[stat_in, vec, vec,
                  pl.BlockSpec((BN_CONV, H, W2, C2), lambda i: (i, 0, 0, 0)),
                  pl.BlockSpec((9, C, C), lambda i: (0, 0, 0))],
        out_specs=(pl.BlockSpec((BN_CONV, H, W2, C2), lambda i: (i, 0, 0, 0)),
                   pl.BlockSpec((1, 2, C2), lambda i: (i, 0, 0))),
        out_shape=(jax.ShapeDtypeStruct((N, H, W2, C2), out_dtype),
                   jax.ShapeDtypeStruct((nb, 2, C2), jnp.float32)),
        scratch_shapes=[pltpu.VMEM((BN_CONV, H + 2, W2 + 1, C2), jnp.bfloat16),
                        pltpu.VMEM((12 * C, C2), jnp.bfloat16)],
        compiler_params=pltpu.CompilerParams(
            dimension_semantics=("parallel",),
            vmem_limit_bytes=100 * 1024 * 1024),
    )(st1, g1, be1, src, w2r)


def _resid_call(x, y, st2, g2, be2, *, N, H, W2, m_total):
    W = 2 * W2
    nbn, nbh = N // BN_RES, H // TH_RES
    nchw = pl.BlockSpec((BN_RES, C, TH_RES, W), lambda ni, hi: (ni, 0, hi, 0))
    band = pl.BlockSpec((BN_RES, TH_RES, W2, C2), lambda ni, hi: (ni, hi, 0, 0))
    vec = pl.BlockSpec((1, C), lambda ni, hi: (0, 0))
    stat_in = pl.BlockSpec(st2.shape, lambda ni, hi: (0, 0, 0))
    return pl.pallas_call(
        functools.partial(_resid_kernel, m_total=m_total),
        grid=(nbn, nbh),
        in_specs=[stat_in, vec, vec, nchw, band],
        out_specs=nchw,
        out_shape=jax.ShapeDtypeStruct((N, C, H, W), jnp.float32),
        compiler_params=pltpu.CompilerParams(
            dimension_semantics=("parallel", "parallel")),
    )(st2, g2, be2, x, y)


def kernel(x, w1, b1, g1, be1, w2, b2, g2, be2):
    del b1, b2
    x = jnp.asarray(x, jnp.float32)
    N, _, H, W = x.shape
    W2 = W // 2
    M = float(N * H * W)
    g1 = jnp.asarray(g1, jnp.float32).reshape(1, C)
    be1 = jnp.asarray(be1, jnp.float32).reshape(1, C)
    g2 = jnp.asarray(g2, jnp.float32).reshape(1, C)
    be2 = jnp.asarray(be2, jnp.float32).reshape(1, C)
    w1r = jnp.asarray(w1, jnp.float32).reshape(9, C, C)
    w2r = jnp.asarray(w2, jnp.float32).reshape(9, C, C)

    y1, st1 = _conv1_call(x, w1r, N=N, H=H, W2=W2, out_dtype=jnp.bfloat16)
    y2, st2 = _conv2_call(y1, st1, w2r, g1, be1, N=N, H=H, W2=W2,
                          m_total=M, out_dtype=jnp.bfloat16)
    return _resid_call(x, y2, st2, g2, be2, N=N, H=H, W2=W2, m_total=M)
